# baseline (speedup 1.0000x reference)
import functools

import jax
import jax.numpy as jnp
from jax import lax
from jax.experimental import pallas as pl
from jax.experimental.pallas import tpu as pltpu
from jax.experimental.pallas import tpu_sc as plsc

N = 16777216
NC = 2
NS = 16
L = 16
NW = NC * NS
PER_W = N // NW
CHUNK = 8192
NCHUNK = PER_W // CHUNK
UNROLL = 8
STEPS = CHUNK // (L * UNROLL)


def _body(err_hbm, unc_hbm, eth_hbm, uth_hbm, num_out, den_out,
          err_v, unc_v, eth_v, uth_v, stage_v):
    wid = lax.axis_index("s") * NC + lax.axis_index("c")
    base = wid * PER_W

    pltpu.sync_copy(eth_hbm, eth_v)
    pltpu.sync_copy(uth_hbm, uth_v)
    eth = eth_v[...]
    uth = uth_v[...]

    def chunk_body(c, carry):
        acc_n, acc_d = carry
        off = base + c * CHUNK
        pltpu.sync_copy(err_hbm.at[pl.ds(off, CHUNK)], err_v)
        pltpu.sync_copy(unc_hbm.at[pl.ds(off, CHUNK)], unc_v)

        def step(i, carry2):
            acc_n, acc_d = carry2
            for j in range(UNROLL):
                o = i * (L * UNROLL) + j * L
                e = err_v[pl.ds(o, L)]
                u = unc_v[pl.ds(o, L)]
                ex = jnp.exp(e * -2.0)
                eu = jnp.exp(u * -2.0)
                low = e <= eth
                cer = u <= uth
                sa = jnp.where(low, ex + ex, 1.0 - ex)
                sb = jnp.where(cer, eu + eu, 1.0 - eu)
                p = (sa * sb) / ((1.0 + ex) * (1.0 + eu))
                acc_d = acc_d + p
                zero = jnp.zeros_like(p)
                acc_n = acc_n + jnp.where(low, jnp.where(cer, p, zero),
                                          jnp.where(cer, zero, p))
            return acc_n, acc_d

        return lax.fori_loop(0, STEPS, step, (acc_n, acc_d))

    zero = jnp.zeros((L,), jnp.float32)
    acc_n, acc_d = lax.fori_loop(0, NCHUNK, chunk_body, (zero, zero))

    stage_v[...] = acc_n
    pltpu.sync_copy(stage_v, num_out.at[wid])
    stage_v[...] = acc_d
    pltpu.sync_copy(stage_v, den_out.at[wid])


@jax.jit
def _partials(error, unc, eth16, uth16):
    mesh = plsc.VectorSubcoreMesh(core_axis_name="c", subcore_axis_name="s")
    f32 = jnp.float32
    run = functools.partial(
        pl.kernel,
        mesh=mesh,
        out_type=[jax.ShapeDtypeStruct((NW, L), f32),
                  jax.ShapeDtypeStruct((NW, L), f32)],
        scratch_types=[
            pltpu.VMEM((CHUNK,), f32),
            pltpu.VMEM((CHUNK,), f32),
            pltpu.VMEM((L,), f32),
            pltpu.VMEM((L,), f32),
            pltpu.VMEM((L,), f32),
        ],
    )(_body)
    return run(error, unc, eth16, uth16)


def kernel(error, unc, error_th, unc_th):
    eth16 = jnp.broadcast_to(error_th.astype(jnp.float32), (L,))
    uth16 = jnp.broadcast_to(unc_th.astype(jnp.float32), (L,))
    num_parts, den_parts = _partials(error, unc, eth16, uth16)
    num = jnp.sum(num_parts)
    den = jnp.sum(den_parts)
    eau = num / (den + 1e-10)
    return -1.0 * jnp.log(eau + 1e-10)


# speedup vs baseline: 1.5914x; 1.5914x over previous
import functools

import jax
import jax.numpy as jnp
from jax import lax
from jax.experimental import pallas as pl
from jax.experimental.pallas import tpu as pltpu
from jax.experimental.pallas import tpu_sc as plsc

N = 16777216
NC = 2
NS = 16
L = 16
NW = NC * NS
PER_W = N // NW
CHUNK = 8192
NCHUNK = PER_W // CHUNK
UNROLL = 8
STEPS = CHUNK // (L * UNROLL)


def _body(err_hbm, unc_hbm, eth_hbm, uth_hbm, num_out, den_out,
          err0_v, err1_v, unc0_v, unc1_v, eth_v, uth_v, stage_v,
          sem_e0, sem_e1, sem_u0, sem_u1):
    wid = lax.axis_index("s") * NC + lax.axis_index("c")
    base = wid * PER_W

    pltpu.sync_copy(eth_hbm, eth_v)
    pltpu.sync_copy(uth_hbm, uth_v)
    eth = eth_v[...]
    uth = uth_v[...]

    bufs = ((err0_v, unc0_v, sem_e0, sem_u0), (err1_v, unc1_v, sem_e1, sem_u1))

    def start(c, b):
        ev, uv, se, su = bufs[b]
        off = base + c * CHUNK
        pltpu.async_copy(err_hbm.at[pl.ds(off, CHUNK)], ev, se)
        pltpu.async_copy(unc_hbm.at[pl.ds(off, CHUNK)], uv, su)

    def wait(b):
        ev, uv, se, su = bufs[b]
        pltpu.make_async_copy(err_hbm.at[pl.ds(0, CHUNK)], ev, se).wait()
        pltpu.make_async_copy(unc_hbm.at[pl.ds(0, CHUNK)], uv, su).wait()

    def compute(b, acc_n, acc_d):
        ev, uv, _, _ = bufs[b]

        def step(i, carry2):
            acc_n, acc_d = carry2
            for j in range(UNROLL):
                o = i * (L * UNROLL) + j * L
                e = ev[pl.ds(o, L)]
                u = uv[pl.ds(o, L)]
                ex = jnp.exp(e * -2.0)
                eu = jnp.exp(u * -2.0)
                low = e <= eth
                cer = u <= uth
                sa = jnp.where(low, ex + ex, 1.0 - ex)
                sb = jnp.where(cer, eu + eu, 1.0 - eu)
                p = (sa * sb) / ((1.0 + ex) * (1.0 + eu))
                acc_d = acc_d + p
                zero = jnp.zeros_like(p)
                acc_n = acc_n + jnp.where(low, jnp.where(cer, p, zero),
                                          jnp.where(cer, zero, p))
            return acc_n, acc_d

        return lax.fori_loop(0, STEPS, step, (acc_n, acc_d))

    start(0, 0)

    def pair_body(it, carry):
        acc_n, acc_d = carry
        c0 = it * 2
        start(c0 + 1, 1)
        wait(0)
        acc_n, acc_d = compute(0, acc_n, acc_d)

        @pl.when(c0 + 2 < NCHUNK)
        def _():
            start(c0 + 2, 0)

        wait(1)
        return compute(1, acc_n, acc_d)

    zero = jnp.zeros((L,), jnp.float32)
    acc_n, acc_d = lax.fori_loop(0, NCHUNK // 2, pair_body, (zero, zero))

    stage_v[...] = acc_n
    pltpu.sync_copy(stage_v, num_out.at[wid])
    stage_v[...] = acc_d
    pltpu.sync_copy(stage_v, den_out.at[wid])


@jax.jit
def _partials(error, unc, eth16, uth16):
    mesh = plsc.VectorSubcoreMesh(core_axis_name="c", subcore_axis_name="s")
    f32 = jnp.float32
    run = functools.partial(
        pl.kernel,
        mesh=mesh,
        out_type=[jax.ShapeDtypeStruct((NW, L), f32),
                  jax.ShapeDtypeStruct((NW, L), f32)],
        scratch_types=[
            pltpu.VMEM((CHUNK,), f32),
            pltpu.VMEM((CHUNK,), f32),
            pltpu.VMEM((CHUNK,), f32),
            pltpu.VMEM((CHUNK,), f32),
            pltpu.VMEM((L,), f32),
            pltpu.VMEM((L,), f32),
            pltpu.VMEM((L,), f32),
            pltpu.SemaphoreType.DMA,
            pltpu.SemaphoreType.DMA,
            pltpu.SemaphoreType.DMA,
            pltpu.SemaphoreType.DMA,
        ],
    )(_body)
    return run(error, unc, eth16, uth16)


def kernel(error, unc, error_th, unc_th):
    eth16 = jnp.broadcast_to(error_th.astype(jnp.float32), (L,))
    uth16 = jnp.broadcast_to(unc_th.astype(jnp.float32), (L,))
    num_parts, den_parts = _partials(error, unc, eth16, uth16)
    num = jnp.sum(num_parts)
    den = jnp.sum(den_parts)
    eau = num / (den + 1e-10)
    return -1.0 * jnp.log(eau + 1e-10)
